# Initial kernel scaffold; baseline (speedup 1.0000x reference)
import jax
import jax.numpy as jnp
from jax.experimental import pallas as pl


def kernel(xyz1, xyz2):
    raise NotImplementedError("write your pallas kernel here")


# speedup vs baseline: 1.4471x; 1.4471x over previous
import jax
import jax.numpy as jnp
from jax import lax
from jax.experimental import pallas as pl

_M_TILE = 512
_BIG_I32 = 2**30


def _chamfer_body(x1_ref, x2t_ref, d1_ref, i1_ref, d2_ref, i2_ref):
    j = pl.program_id(1)
    x1 = x1_ref[0]
    x2 = x2t_ref[0]

    dx = x1[:, 0:1] - x2[0:1, :]
    dy = x1[:, 1:2] - x2[1:2, :]
    dz = x1[:, 2:3] - x2[2:3, :]
    d = dx * dx + dy * dy + dz * dz

    rmin = jnp.min(d, axis=1, keepdims=True)
    col_ids = lax.broadcasted_iota(jnp.int32, d.shape, 1) + j * _M_TILE
    ridx = jnp.min(jnp.where(d == rmin, col_ids, _BIG_I32), axis=1,
                   keepdims=True)

    @pl.when(j == 0)
    def _init():
        d1_ref[0] = rmin
        i1_ref[0] = ridx

    @pl.when(j > 0)
    def _merge():
        prev = d1_ref[0]
        upd = rmin < prev
        d1_ref[0] = jnp.where(upd, rmin, prev)
        i1_ref[0] = jnp.where(upd, ridx, i1_ref[0])

    cmin = jnp.min(d, axis=0, keepdims=True)
    row_ids = lax.broadcasted_iota(jnp.int32, d.shape, 0)
    cidx = jnp.min(jnp.where(d == cmin, row_ids, _BIG_I32), axis=0,
                   keepdims=True)
    d2_ref[0] = cmin
    i2_ref[0] = cidx


def kernel(xyz1, xyz2):
    B, N, _ = xyz1.shape
    M = xyz2.shape[1]
    xyz2t = xyz2.transpose(0, 2, 1)
    n_tiles = M // _M_TILE

    grid = (B, n_tiles)
    d1, i1, d2, i2 = pl.pallas_call(
        _chamfer_body,
        grid=grid,
        in_specs=[
            pl.BlockSpec((1, N, 3), lambda b, j: (b, 0, 0)),
            pl.BlockSpec((1, 3, _M_TILE), lambda b, j: (b, 0, j)),
        ],
        out_specs=[
            pl.BlockSpec((1, N, 1), lambda b, j: (b, 0, 0)),
            pl.BlockSpec((1, N, 1), lambda b, j: (b, 0, 0)),
            pl.BlockSpec((1, 1, _M_TILE), lambda b, j: (b, 0, j)),
            pl.BlockSpec((1, 1, _M_TILE), lambda b, j: (b, 0, j)),
        ],
        out_shape=[
            jax.ShapeDtypeStruct((B, N, 1), jnp.float32),
            jax.ShapeDtypeStruct((B, N, 1), jnp.int32),
            jax.ShapeDtypeStruct((B, 1, M), jnp.float32),
            jax.ShapeDtypeStruct((B, 1, M), jnp.int32),
        ],
    )(xyz1, xyz2t)

    dist1 = d1.reshape(B, N)
    idx1 = i1.reshape(B, N)
    dist2 = d2.reshape(B, M)
    idx2 = i2.reshape(B, M)
    return (dist1, dist2, idx1, idx2)


# speedup vs baseline: 1.6825x; 1.1626x over previous
import jax
import jax.numpy as jnp
from jax import lax
from jax.experimental import pallas as pl

_M_TILE = 512
_BIG_F32 = 1e9


def _chamfer_body(x1_ref, x2t_ref, d1_ref, i1_ref, d2_ref, i2_ref):
    j = pl.program_id(1)
    x1 = x1_ref[0]
    x2 = x2t_ref[0]

    dx = x1[:, 0:1] - x2[0:1, :]
    dy = x1[:, 1:2] - x2[1:2, :]
    dz = x1[:, 2:3] - x2[2:3, :]
    d = dx * dx + dy * dy + dz * dz

    n = d.shape[0]
    rmin = jnp.min(d, axis=1, keepdims=True)
    col_ids = lax.broadcasted_iota(jnp.int32, (1, _M_TILE), 1).astype(jnp.float32)
    ridx_f = jnp.min(jnp.where(d == rmin, col_ids, _BIG_F32), axis=1,
                     keepdims=True)
    ridx = ridx_f.astype(jnp.int32) + j * _M_TILE

    @pl.when(j == 0)
    def _init():
        d1_ref[0] = rmin
        i1_ref[0] = ridx

    @pl.when(j > 0)
    def _merge():
        prev = d1_ref[0]
        upd = rmin < prev
        d1_ref[0] = jnp.where(upd, rmin, prev)
        i1_ref[0] = jnp.where(upd, ridx, i1_ref[0])

    cmin = jnp.min(d, axis=0, keepdims=True)
    row_ids = lax.broadcasted_iota(jnp.int32, (n, 1), 0).astype(jnp.float32)
    cidx_f = jnp.min(jnp.where(d == cmin, row_ids, _BIG_F32), axis=0,
                     keepdims=True)
    d2_ref[0] = cmin
    i2_ref[0] = cidx_f.astype(jnp.int32)


def kernel(xyz1, xyz2):
    B, N, _ = xyz1.shape
    M = xyz2.shape[1]
    xyz2t = xyz2.transpose(0, 2, 1)
    n_tiles = M // _M_TILE

    grid = (B, n_tiles)
    d1, i1, d2, i2 = pl.pallas_call(
        _chamfer_body,
        grid=grid,
        in_specs=[
            pl.BlockSpec((1, N, 3), lambda b, j: (b, 0, 0)),
            pl.BlockSpec((1, 3, _M_TILE), lambda b, j: (b, 0, j)),
        ],
        out_specs=[
            pl.BlockSpec((1, N, 1), lambda b, j: (b, 0, 0)),
            pl.BlockSpec((1, N, 1), lambda b, j: (b, 0, 0)),
            pl.BlockSpec((1, 1, _M_TILE), lambda b, j: (b, 0, j)),
            pl.BlockSpec((1, 1, _M_TILE), lambda b, j: (b, 0, j)),
        ],
        out_shape=[
            jax.ShapeDtypeStruct((B, N, 1), jnp.float32),
            jax.ShapeDtypeStruct((B, N, 1), jnp.int32),
            jax.ShapeDtypeStruct((B, 1, M), jnp.float32),
            jax.ShapeDtypeStruct((B, 1, M), jnp.int32),
        ],
    )(xyz1, xyz2t)

    dist1 = d1.reshape(B, N)
    idx1 = i1.reshape(B, N)
    dist2 = d2.reshape(B, M)
    idx2 = i2.reshape(B, M)
    return (dist1, dist2, idx1, idx2)


# speedup vs baseline: 2.1010x; 1.2487x over previous
import jax
import jax.numpy as jnp
from jax import lax
from jax.experimental import pallas as pl

_M_TILE = 2048
_BIG_F32 = 1e9


def _chamfer_body(x1_ref, x2t_ref, d1_ref, i1_ref, d2_ref, i2_ref):
    j = pl.program_id(1)
    x1 = x1_ref[0]
    x2 = x2t_ref[0]

    dx = x1[:, 0:1] - x2[0:1, :]
    dy = x1[:, 1:2] - x2[1:2, :]
    dz = x1[:, 2:3] - x2[2:3, :]
    d = dx * dx + dy * dy + dz * dz

    n = d.shape[0]
    rmin = jnp.min(d, axis=1, keepdims=True)
    col_ids = lax.broadcasted_iota(jnp.int32, (1, _M_TILE), 1).astype(jnp.float32)
    ridx_f = jnp.min(jnp.where(d == rmin, col_ids, _BIG_F32), axis=1,
                     keepdims=True)
    ridx = ridx_f.astype(jnp.int32) + j * _M_TILE

    @pl.when(j == 0)
    def _init():
        d1_ref[0] = rmin
        i1_ref[0] = ridx

    @pl.when(j > 0)
    def _merge():
        prev = d1_ref[0]
        upd = rmin < prev
        d1_ref[0] = jnp.where(upd, rmin, prev)
        i1_ref[0] = jnp.where(upd, ridx, i1_ref[0])

    cmin = jnp.min(d, axis=0, keepdims=True)
    row_ids = lax.broadcasted_iota(jnp.int32, (n, 1), 0).astype(jnp.float32)
    cidx_f = jnp.min(jnp.where(d == cmin, row_ids, _BIG_F32), axis=0,
                     keepdims=True)
    d2_ref[0] = cmin
    i2_ref[0] = cidx_f.astype(jnp.int32)


def kernel(xyz1, xyz2):
    B, N, _ = xyz1.shape
    M = xyz2.shape[1]
    xyz2t = xyz2.transpose(0, 2, 1)
    n_tiles = M // _M_TILE

    grid = (B, n_tiles)
    d1, i1, d2, i2 = pl.pallas_call(
        _chamfer_body,
        grid=grid,
        in_specs=[
            pl.BlockSpec((1, N, 3), lambda b, j: (b, 0, 0)),
            pl.BlockSpec((1, 3, _M_TILE), lambda b, j: (b, 0, j)),
        ],
        out_specs=[
            pl.BlockSpec((1, N, 1), lambda b, j: (b, 0, 0)),
            pl.BlockSpec((1, N, 1), lambda b, j: (b, 0, 0)),
            pl.BlockSpec((1, 1, _M_TILE), lambda b, j: (b, 0, j)),
            pl.BlockSpec((1, 1, _M_TILE), lambda b, j: (b, 0, j)),
        ],
        out_shape=[
            jax.ShapeDtypeStruct((B, N, 1), jnp.float32),
            jax.ShapeDtypeStruct((B, N, 1), jnp.int32),
            jax.ShapeDtypeStruct((B, 1, M), jnp.float32),
            jax.ShapeDtypeStruct((B, 1, M), jnp.int32),
        ],
    )(xyz1, xyz2t)

    dist1 = d1.reshape(B, N)
    idx1 = i1.reshape(B, N)
    dist2 = d2.reshape(B, M)
    idx2 = i2.reshape(B, M)
    return (dist1, dist2, idx1, idx2)
